# baseline (speedup 1.0000x reference)
import functools

import jax
import jax.numpy as jnp
from jax import lax
from jax.experimental import pallas as pl
from jax.experimental.pallas import tpu as pltpu
from jax.experimental.pallas import tpu_sc as plsc

N = 10000
D = 128
T = 16
NP = 10240
RT = NP // 32
DROW = RT
ACC_R = RT + 8
EG = 2560
EPAD = EG * 128
CH = 32
NCH = EG // CH
CAP = 12288
CAPG = CAP // 128
XG = 96
BLK = 1024
CAP2 = 16384
VEC2 = CAP2 // 16
G2 = CAP2 // 128
SB = 20
SBC = 1040
ZROW = N + 200

_SEL = 32 * CAP


def _mesh():
    return plsc.VectorSubcoreMesh(
        core_axis_name="c", subcore_axis_name="s",
        num_cores=2, num_subcores=16)


@functools.cache
def _sc_embed_kernel():
    return pl.kernel(
        _sc_embed_body,
        out_type=jax.ShapeDtypeStruct((XG * 128, D), jnp.float32),
        mesh=_mesh(),
        scratch_types=[
            pltpu.VMEM((3, 128), jnp.int32),
            pltpu.VMEM((128, D), jnp.float32),
            pltpu.SemaphoreType.DMA,
        ],
    )


def _sc_embed_body(table_hbm, xi_hbm, hraw_hbm, xi_v, rows_v, sem):
    c = lax.axis_index("c")
    s = lax.axis_index("s")
    wid = c * 16 + s
    pltpu.sync_copy(xi_hbm.at[wid], xi_v)
    for j in range(3):
        pltpu.async_copy(table_hbm.at[xi_v.at[j]], rows_v, sem).wait()
        pltpu.sync_copy(rows_v, hraw_hbm.at[pl.ds((wid * 3 + j) * 128, 128)])


@functools.cache
def _sc_prep_kernel():
    return pl.kernel(
        _sc_prep_body,
        out_type=(jax.ShapeDtypeStruct((_SEL,), jnp.int32),
                  jax.ShapeDtypeStruct((_SEL,), jnp.int32),
                  jax.ShapeDtypeStruct((_SEL,), jnp.int32),
                  jax.ShapeDtypeStruct((NP * T,), jnp.float32),
                  jax.ShapeDtypeStruct((32 * CAP2,), jnp.int32),
                  jax.ShapeDtypeStruct((32 * VEC2,), jnp.int32)),
        mesh=_mesh(),
        compiler_params=pltpu.CompilerParams(needs_layout_passes=False),
        scratch_types=[
            pltpu.VMEM((CH * 128,), jnp.int32),
            pltpu.VMEM((CH * 128,), jnp.int32),
            pltpu.VMEM((CH * 128,), jnp.int32),
            pltpu.VMEM((CAP + 16,), jnp.int32),
            pltpu.VMEM((CAP + 16,), jnp.int32),
            pltpu.VMEM((CAP + 16,), jnp.int32),
            pltpu.VMEM((ACC_R * T,), jnp.float32),
            pltpu.VMEM((SB * SBC,), jnp.int32),
            pltpu.VMEM((SB * SBC,), jnp.int32),
            pltpu.VMEM((CAP2 + 16,), jnp.int32),
            pltpu.VMEM((VEC2 + 16,), jnp.int32),
        ],
    )


def _sc_prep_body(src_hbm, dst_hbm, typ_hbm,
                  sels_hbm, seld_hbm, selt_hbm, deg_hbm, sels2_hbm, rowd_hbm,
                  src_v, dst_v, typ_v, sels_v, seld_v, selt_v, deg_l,
                  sb_src, sb_dst, sels2_v, rowd_v):
    c = lax.axis_index("c")
    s = lax.axis_index("s")
    wid = c * 16 + s
    lo = wid * RT

    zv = jnp.zeros((16,), jnp.int32)
    dv = jnp.full((16,), DROW, jnp.int32)

    def fill(i, carry):
        sels_v[pl.ds(i * 16, 16)] = zv
        seld_v[pl.ds(i * 16, 16)] = dv
        selt_v[pl.ds(i * 16, 16)] = zv
        return carry

    lax.fori_loop(0, (CAP + 16) // 16, fill, 0)

    zf = jnp.zeros((16,), jnp.float32)

    def zrow(i, carry):
        deg_l[pl.ds(i * 16, 16)] = zf
        return carry

    lax.fori_loop(0, ACC_R * T // 16, zrow, 0)

    def chunk(ci, cur):
        pltpu.sync_copy(src_hbm.at[pl.ds(ci * CH * 128, CH * 128)], src_v)
        pltpu.sync_copy(dst_hbm.at[pl.ds(ci * CH * 128, CH * 128)], dst_v)
        pltpu.sync_copy(typ_hbm.at[pl.ds(ci * CH * 128, CH * 128)], typ_v)
        for v in range(CH * 8):
            dsts = dst_v[pl.ds(v * 16, 16)]
            srcs = src_v[pl.ds(v * 16, 16)]
            typs = typ_v[pl.ds(v * 16, 16)]
            m = (dsts >= lo) & (dsts < lo + RT)
            plsc.store_compressed(sels_v.at[pl.ds(cur, 16)], srcs, mask=m)
            plsc.store_compressed(seld_v.at[pl.ds(cur, 16)], dsts - lo, mask=m)
            plsc.store_compressed(selt_v.at[pl.ds(cur, 16)], typs, mask=m)
            cnt = plsc.all_reduce_population_count(m)[0]
            cur = cur + cnt
        return cur

    lax.fori_loop(0, NCH, chunk, jnp.int32(0))

    e0 = jnp.where(lax.iota(jnp.int32, 16) == 0, 1.0, 0.0)

    def dbody(i, carry):
        dvec = seld_v[pl.ds(i * 16, 16)]
        for k in range(16):
            d = dvec[k]
            deg_l[pl.ds(d * T, 16)] = deg_l[pl.ds(d * T, 16)] + e0
        return carry

    lax.fori_loop(0, CAP // 16, dbody, 0)

    sent = jnp.full((16,), 30000, jnp.int32)
    zrowv = jnp.full((16,), ZROW, jnp.int32)
    drowv = jnp.full((16,), DROW, jnp.int32)

    def sfill(i, carry):
        sb_dst[pl.ds(i * 16, 16)] = sent
        return carry

    lax.fori_loop(0, SB * SBC // 16, sfill, 0)

    def s2fill(i, carry):
        sels2_v[pl.ds(i * 16, 16)] = zrowv
        return carry

    lax.fori_loop(0, (CAP2 + 16) // 16, s2fill, 0)

    def rdfill(i, carry):
        rowd_v[pl.ds(i * 16, 16)] = drowv
        return carry

    lax.fori_loop(0, (VEC2 + 16) // 16, rdfill, 0)

    def l1_outer(b, carry):
        def l1_inner(v, cur):
            dv = seld_v[pl.ds(v * 16, 16)]
            sv = sels_v[pl.ds(v * 16, 16)]
            m = (dv >= b * 16) & (dv < b * 16 + 16)
            plsc.store_compressed(sb_src.at[pl.ds(cur, 16)], sv, mask=m)
            plsc.store_compressed(sb_dst.at[pl.ds(cur, 16)], dv, mask=m)
            return cur + plsc.all_reduce_population_count(m)[0]

        lax.fori_loop(0, CAP // 16, l1_inner, b * SBC)
        return carry

    lax.fori_loop(0, SB, l1_outer, 0)

    lanes16 = lax.iota(jnp.int32, 16)

    def l2_outer(r, carry):
        cur2, curv = carry
        vb = (r // 16) * SBC

        def l2_inner(v, c2):
            dv = sb_dst[pl.ds(vb + v * 16, 16)]
            sv = sb_src[pl.ds(vb + v * 16, 16)]
            m = dv == r
            plsc.store_compressed(sels2_v.at[pl.ds(c2, 16)], sv, mask=m)
            return c2 + plsc.all_reduce_population_count(m)[0]

        c2 = lax.fori_loop(0, SBC // 16, l2_inner, cur2)
        padn = (16 - (c2 & 15)) & 15
        plsc.store_compressed(sels2_v.at[pl.ds(c2, 16)], zrowv,
                              mask=lanes16 < padn)
        c2 = c2 + padn
        nvec = (c2 - cur2) >> 4
        plsc.store_compressed(rowd_v.at[pl.ds(curv, 16)],
                              jnp.full((16,), 0, jnp.int32) + r,
                              mask=lanes16 < nvec)
        return (c2, curv + nvec)

    lax.fori_loop(0, RT, l2_outer, (jnp.int32(0), jnp.int32(0)))

    pltpu.sync_copy(sels_v.at[pl.ds(0, CAP)], sels_hbm.at[pl.ds(wid * CAP, CAP)])
    pltpu.sync_copy(seld_v.at[pl.ds(0, CAP)], seld_hbm.at[pl.ds(wid * CAP, CAP)])
    pltpu.sync_copy(selt_v.at[pl.ds(0, CAP)], selt_hbm.at[pl.ds(wid * CAP, CAP)])
    pltpu.sync_copy(deg_l.at[pl.ds(0, RT * T)], deg_hbm.at[pl.ds(lo * T, RT * T)])
    pltpu.sync_copy(sels2_v.at[pl.ds(0, CAP2)],
                    sels2_hbm.at[pl.ds(wid * CAP2, CAP2)])
    pltpu.sync_copy(rowd_v.at[pl.ds(0, VEC2)],
                    rowd_hbm.at[pl.ds(wid * VEC2, VEC2)])


@functools.cache
def _sc_u_kernel():
    return pl.kernel(
        _sc_u_body,
        out_type=jax.ShapeDtypeStruct((NP, T), jnp.float32),
        mesh=_mesh(),
        scratch_types=[
            pltpu.VMEM((NP,), jnp.float32),
            pltpu.VMEM((CAP,), jnp.int32),
            pltpu.VMEM((CAP,), jnp.int32),
            pltpu.VMEM((CAP,), jnp.int32),
            pltpu.VMEM((ACC_R, T), jnp.float32),
        ],
    )


def _sc_u_body(r_hbm, sels_hbm, seld_hbm, selt_hbm,
               u_hbm,
               r_v, sels_v, seld_v, selt_v, u_l):
    c = lax.axis_index("c")
    s = lax.axis_index("s")
    wid = c * 16 + s
    lo = wid * RT
    pltpu.sync_copy(r_hbm, r_v)
    pltpu.sync_copy(sels_hbm.at[pl.ds(wid * CAP, CAP)], sels_v)
    pltpu.sync_copy(seld_hbm.at[pl.ds(wid * CAP, CAP)], seld_v)
    pltpu.sync_copy(selt_hbm.at[pl.ds(wid * CAP, CAP)], selt_v)

    zf = jnp.zeros((16,), jnp.float32)

    def zrow(i, carry):
        u_l[i, :] = zf
        return carry

    lax.fori_loop(0, ACC_R, zrow, 0)

    lanes = lax.iota(jnp.int32, 16)

    def body(i, carry):
        svec = sels_v[pl.ds(i * 16, 16)]
        dvec = seld_v[pl.ds(i * 16, 16)]
        tvec = selt_v[pl.ds(i * 16, 16)]
        for k in range(16):
            rs = r_v[pl.ds(svec[k], 16)][0]
            u_l[dvec[k], :] = (u_l[dvec[k], :]
                               + jnp.where(lanes == tvec[k], rs, 0.0))
        return carry

    lax.fori_loop(0, CAP // 16, body, 0)
    pltpu.sync_copy(u_l.at[pl.ds(0, RT)], u_hbm.at[pl.ds(lo, RT)])


@functools.cache
def _sc_spmm_kernel():
    return pl.kernel(
        _sc_spmm_body,
        out_type=jax.ShapeDtypeStruct((NP, D), jnp.float32),
        mesh=_mesh(),
        scratch_types=[
            pltpu.VMEM((G2, 128), jnp.int32),
            pltpu.VMEM((VEC2 + 16,), jnp.int32),
            pltpu.VMEM((2, 128, D), jnp.float32),
            pltpu.VMEM((ACC_R, D), jnp.float32),
            pltpu.SemaphoreType.DMA,
            pltpu.SemaphoreType.DMA,
        ],
    )


def _sc_spmm_body(g_hbm, sels2_hbm, rowd_hbm, z128_hbm,
                  p0_hbm,
                  sels_v, rowd_v, rows_v, acc, sem0, sem1):
    c = lax.axis_index("c")
    s = lax.axis_index("s")
    wid = c * 16 + s
    lo = wid * RT
    pltpu.sync_copy(sels2_hbm.at[pl.ds(wid * G2, G2)], sels_v)
    pltpu.sync_copy(rowd_hbm.at[pl.ds(wid * VEC2, VEC2)],
                    rowd_v.at[pl.ds(0, VEC2)])
    pltpu.sync_copy(z128_hbm, acc.at[pl.ds(0, 128)])
    pltpu.sync_copy(z128_hbm, acc.at[pl.ds(128, 128)])
    pltpu.sync_copy(z128_hbm.at[pl.ds(0, ACC_R - 256)],
                    acc.at[pl.ds(256, ACC_R - 256)])

    def issue(k, buf, sem):
        pltpu.async_copy(g_hbm.at[sels_v.at[k]], rows_v.at[buf], sem)

    def drain(buf, sem):
        pltpu.make_async_copy(g_hbm.at[sels_v.at[0]],
                              rows_v.at[buf], sem).wait()

    def accum(k, buf):
        dvec8 = rowd_v[pl.ds(k * 8, 16)]
        for w in range(8):
            d = dvec8[w]
            for p in range(8):
                v = [rows_v[buf, w * 16 + i, pl.ds(p * 16, 16)]
                     for i in range(16)]
                while len(v) > 1:
                    v = [v[2 * j] + v[2 * j + 1] for j in range(len(v) // 2)]
                plsc.addupdate(acc.at[d, pl.ds(p * 16, 16)], v[0])

    issue(0, 0, sem0)

    def body(m, carry):
        k0 = 2 * m
        drain(0, sem0)
        issue(k0 + 1, 1, sem1)
        accum(k0, 0)
        drain(1, sem1)

        @pl.when(k0 + 2 < G2)
        def _():
            issue(k0 + 2, 0, sem0)

        accum(k0 + 1, 1)
        return carry

    lax.fori_loop(0, G2 // 2, body, 0)
    pltpu.sync_copy(acc.at[pl.ds(0, RT)], p0_hbm.at[pl.ds(lo, RT)])


def _row_mask():
    rows = (pl.program_id(0) * BLK
            + lax.broadcasted_iota(jnp.int32, (BLK, 1), 0))
    return rows < N


def _tc_prep_body(h_ref, deg16_ref, r_ref, g0_ref):
    hr = h_ref[...]
    deg = jnp.maximum(deg16_ref[...][:, 0], 1.0)
    r = lax.rsqrt(deg)
    nrm = jnp.sqrt(jnp.sum(hr * hr, axis=1, keepdims=True))
    h0 = hr * jnp.minimum(1.0, 1.0 / (nrm + 1e-7))
    r_ref[...] = r
    g0_ref[...] = jnp.where(_row_mask(), h0 * r[:, None], 0.0)


def _tc_prep(hraw, deg16):
    grid = NP // BLK
    return pl.pallas_call(
        _tc_prep_body,
        grid=(grid,),
        in_specs=[pl.BlockSpec((BLK, D), lambda i: (i, 0)),
                  pl.BlockSpec((BLK, T), lambda i: (i, 0))],
        out_specs=[pl.BlockSpec((BLK,), lambda i: (i,)),
                   pl.BlockSpec((BLK, D), lambda i: (i, 0))],
        out_shape=[jax.ShapeDtypeStruct((NP,), jnp.float32),
                   jax.ShapeDtypeStruct((NP, D), jnp.float32)],
    )(hraw, deg16)


def _tc_c_body(u_ref, r_ref, et_ref, we1_ref, b1_ref, we2_ref, b2_ref,
               c1_ref, c2_ref):
    hi = lax.Precision.HIGHEST
    v1 = jnp.dot(et_ref[...], we1_ref[...], precision=hi,
                 preferred_element_type=jnp.float32) + b1_ref[...][None, :]
    v2 = jnp.dot(et_ref[...], we2_ref[...], precision=hi,
                 preferred_element_type=jnp.float32) + b2_ref[...][None, :]
    u = u_ref[...]
    r = r_ref[...][:, None]
    c1_ref[...] = jnp.dot(u, v1, precision=hi,
                          preferred_element_type=jnp.float32) * r
    c2_ref[...] = jnp.dot(u, v2, precision=hi,
                          preferred_element_type=jnp.float32) * r


def _tc_c(u, r, et, we1, b1, we2, b2):
    grid = NP // BLK
    return pl.pallas_call(
        _tc_c_body,
        grid=(grid,),
        in_specs=[pl.BlockSpec((BLK, T), lambda i: (i, 0)),
                  pl.BlockSpec((BLK,), lambda i: (i,)),
                  pl.BlockSpec((T, T), lambda i: (0, 0)),
                  pl.BlockSpec((T, D), lambda i: (0, 0)),
                  pl.BlockSpec((D,), lambda i: (0,)),
                  pl.BlockSpec((T, D), lambda i: (0, 0)),
                  pl.BlockSpec((D,), lambda i: (0,))],
        out_specs=[pl.BlockSpec((BLK, D), lambda i: (i, 0)),
                   pl.BlockSpec((BLK, D), lambda i: (i, 0))],
        out_shape=[jax.ShapeDtypeStruct((NP, D), jnp.float32),
                   jax.ShapeDtypeStruct((NP, D), jnp.float32)],
    )(u, r, et, we1, b1, we2, b2)


def _tc_layer_body(p0_ref, r_ref, c_ref, w_ref, out_ref, *, act, emit_g):
    r = r_ref[...][:, None]
    accv = p0_ref[...] * r
    z = jnp.dot(accv, w_ref[...], precision=lax.Precision.HIGHEST,
                preferred_element_type=jnp.float32) + c_ref[...]
    if act:
        z = jnp.where(z > 0.0, z, jnp.exp(jnp.minimum(z, 0.0)) - 1.0)
    if emit_g:
        z = jnp.where(_row_mask(), z * r, 0.0)
    out_ref[...] = z


def _tc_layer(p0, r, cc, wx, act, emit_g):
    grid = NP // BLK
    return pl.pallas_call(
        functools.partial(_tc_layer_body, act=act, emit_g=emit_g),
        grid=(grid,),
        in_specs=[pl.BlockSpec((BLK, D), lambda i: (i, 0)),
                  pl.BlockSpec((BLK,), lambda i: (i,)),
                  pl.BlockSpec((BLK, D), lambda i: (i, 0)),
                  pl.BlockSpec((D, D), lambda i: (0, 0))],
        out_specs=pl.BlockSpec((BLK, D), lambda i: (i, 0)),
        out_shape=jax.ShapeDtypeStruct((NP, D), jnp.float32),
    )(p0, r, cc, wx)


def kernel(x, edge_index, edge_attr, node_table, edge_table,
           W1, b1, W2, b2, slices):
    f32 = jnp.float32
    src = edge_index[0].astype(jnp.int32)
    dst = edge_index[1].astype(jnp.int32)
    typ = edge_attr[:, 0].astype(jnp.int32)
    xi = x[:, 0].astype(jnp.int32)
    e = src.shape[0]
    src_p = jnp.concatenate([src, jnp.zeros((EPAD - e,), jnp.int32)])
    dst_p = jnp.concatenate([dst, jnp.full((EPAD - e,), NP, jnp.int32)])
    typ_p = jnp.concatenate([typ, jnp.zeros((EPAD - e,), jnp.int32)])
    xi_p = jnp.concatenate(
        [xi, jnp.zeros((XG * 128 - N,), jnp.int32)]).reshape(32, XG // 32, 128)
    z128 = jnp.zeros((128, D), f32)

    hraw = _sc_embed_kernel()(node_table, xi_p)
    sels, seld, selt, degf, sels2, rowd = _sc_prep_kernel()(src_p, dst_p, typ_p)
    r, g = _tc_prep(hraw[:NP], degf.reshape(NP, T))
    u = _sc_u_kernel()(r, sels, seld, selt)
    c1, c2 = _tc_c(u, r, edge_table, W1[D:], b1, W2[D:], b2)
    wx1, wx2 = W1[:D], W2[:D]

    h = g
    for layer in range(6):
        p0 = _sc_spmm_kernel()(g, sels2.reshape(32 * G2, 128), rowd, z128)
        if layer % 2 == 0:
            g = _tc_layer(p0, r, c1, wx1, act=True, emit_g=True)
        elif layer < 5:
            g = _tc_layer(p0, r, c2, wx2, act=False, emit_g=True)
        else:
            h = _tc_layer(p0, r, c2, wx2, act=False, emit_g=False)

    out = h[:N].reshape(N // 1000, 1000, D)
    return out * jnp.asarray(slices // 1000, dtype=out.dtype)


# speedup vs baseline: 1.0003x; 1.0003x over previous
import functools

import jax
import jax.numpy as jnp
from jax import lax
from jax.experimental import pallas as pl
from jax.experimental.pallas import tpu as pltpu
from jax.experimental.pallas import tpu_sc as plsc

N = 10000
D = 128
T = 16
NP = 10240
RT = NP // 32
DROW = RT
ACC_R = RT + 8
EG = 2560
EPAD = EG * 128
CH = 32
NCH = EG // CH
CAP = 12288
CAPG = CAP // 128
XG = 96
BLK = 1024
CAP2 = 16384
VEC2 = CAP2 // 16
G2 = CAP2 // 128
SB = 20
SBC = 1040
ZROW = N + 200

_SEL = 32 * CAP


def _mesh():
    return plsc.VectorSubcoreMesh(
        core_axis_name="c", subcore_axis_name="s",
        num_cores=2, num_subcores=16)


@functools.cache
def _sc_embed_kernel():
    return pl.kernel(
        _sc_embed_body,
        out_type=jax.ShapeDtypeStruct((XG * 128, D), jnp.float32),
        mesh=_mesh(),
        scratch_types=[
            pltpu.VMEM((3, 128), jnp.int32),
            pltpu.VMEM((128, D), jnp.float32),
            pltpu.SemaphoreType.DMA,
        ],
    )


def _sc_embed_body(table_hbm, xi_hbm, hraw_hbm, xi_v, rows_v, sem):
    c = lax.axis_index("c")
    s = lax.axis_index("s")
    wid = c * 16 + s
    pltpu.sync_copy(xi_hbm.at[wid], xi_v)
    for j in range(3):
        pltpu.async_copy(table_hbm.at[xi_v.at[j]], rows_v, sem).wait()
        pltpu.sync_copy(rows_v, hraw_hbm.at[pl.ds((wid * 3 + j) * 128, 128)])


@functools.cache
def _sc_prep_kernel():
    return pl.kernel(
        _sc_prep_body,
        out_type=(jax.ShapeDtypeStruct((_SEL,), jnp.int32),
                  jax.ShapeDtypeStruct((_SEL,), jnp.int32),
                  jax.ShapeDtypeStruct((_SEL,), jnp.int32),
                  jax.ShapeDtypeStruct((NP * T,), jnp.float32),
                  jax.ShapeDtypeStruct((32 * CAP2,), jnp.int32),
                  jax.ShapeDtypeStruct((32 * VEC2,), jnp.int32)),
        mesh=_mesh(),
        compiler_params=pltpu.CompilerParams(needs_layout_passes=False),
        scratch_types=[
            pltpu.VMEM((CH * 128,), jnp.int32),
            pltpu.VMEM((CH * 128,), jnp.int32),
            pltpu.VMEM((CH * 128,), jnp.int32),
            pltpu.VMEM((CAP + 16,), jnp.int32),
            pltpu.VMEM((CAP + 16,), jnp.int32),
            pltpu.VMEM((CAP + 16,), jnp.int32),
            pltpu.VMEM((ACC_R * T,), jnp.float32),
            pltpu.VMEM((SB * SBC,), jnp.int32),
            pltpu.VMEM((SB * SBC,), jnp.int32),
            pltpu.VMEM((CAP2 + 16,), jnp.int32),
            pltpu.VMEM((VEC2 + 16,), jnp.int32),
        ],
    )


def _sc_prep_body(src_hbm, dst_hbm, typ_hbm,
                  sels_hbm, seld_hbm, selt_hbm, deg_hbm, sels2_hbm, rowd_hbm,
                  src_v, dst_v, typ_v, sels_v, seld_v, selt_v, deg_l,
                  sb_src, sb_dst, sels2_v, rowd_v):
    c = lax.axis_index("c")
    s = lax.axis_index("s")
    wid = c * 16 + s
    lo = wid * RT

    zv = jnp.zeros((16,), jnp.int32)
    dv = jnp.full((16,), DROW, jnp.int32)

    def fill(i, carry):
        sels_v[pl.ds(i * 16, 16)] = zv
        seld_v[pl.ds(i * 16, 16)] = dv
        selt_v[pl.ds(i * 16, 16)] = zv
        return carry

    lax.fori_loop(0, (CAP + 16) // 16, fill, 0)

    zf = jnp.zeros((16,), jnp.float32)

    def zrow(i, carry):
        deg_l[pl.ds(i * 16, 16)] = zf
        return carry

    lax.fori_loop(0, ACC_R * T // 16, zrow, 0)

    def chunk(ci, cur):
        pltpu.sync_copy(src_hbm.at[pl.ds(ci * CH * 128, CH * 128)], src_v)
        pltpu.sync_copy(dst_hbm.at[pl.ds(ci * CH * 128, CH * 128)], dst_v)
        pltpu.sync_copy(typ_hbm.at[pl.ds(ci * CH * 128, CH * 128)], typ_v)
        for v in range(CH * 8):
            dsts = dst_v[pl.ds(v * 16, 16)]
            srcs = src_v[pl.ds(v * 16, 16)]
            typs = typ_v[pl.ds(v * 16, 16)]
            m = (dsts >= lo) & (dsts < lo + RT)
            plsc.store_compressed(sels_v.at[pl.ds(cur, 16)], srcs, mask=m)
            plsc.store_compressed(seld_v.at[pl.ds(cur, 16)], dsts - lo, mask=m)
            plsc.store_compressed(selt_v.at[pl.ds(cur, 16)], typs, mask=m)
            cnt = plsc.all_reduce_population_count(m)[0]
            cur = cur + cnt
        return cur

    lax.fori_loop(0, NCH, chunk, jnp.int32(0))

    e0 = jnp.where(lax.iota(jnp.int32, 16) == 0, 1.0, 0.0)

    def dbody(i, carry):
        dvec = seld_v[pl.ds(i * 16, 16)]
        for k in range(16):
            d = dvec[k]
            deg_l[pl.ds(d * T, 16)] = deg_l[pl.ds(d * T, 16)] + e0
        return carry

    lax.fori_loop(0, CAP // 16, dbody, 0)

    sent = jnp.full((16,), 30000, jnp.int32)
    zrowv = jnp.full((16,), ZROW, jnp.int32)
    drowv = jnp.full((16,), DROW, jnp.int32)

    def sfill(i, carry):
        sb_dst[pl.ds(i * 16, 16)] = sent
        return carry

    lax.fori_loop(0, SB * SBC // 16, sfill, 0)

    def s2fill(i, carry):
        sels2_v[pl.ds(i * 16, 16)] = zrowv
        return carry

    lax.fori_loop(0, (CAP2 + 16) // 16, s2fill, 0)

    def rdfill(i, carry):
        rowd_v[pl.ds(i * 16, 16)] = drowv
        return carry

    lax.fori_loop(0, (VEC2 + 16) // 16, rdfill, 0)

    def l1_outer(b, carry):
        def l1_inner(v, cur):
            dv = seld_v[pl.ds(v * 16, 16)]
            sv = sels_v[pl.ds(v * 16, 16)]
            m = (dv >= b * 16) & (dv < b * 16 + 16)
            plsc.store_compressed(sb_src.at[pl.ds(cur, 16)], sv, mask=m)
            plsc.store_compressed(sb_dst.at[pl.ds(cur, 16)], dv, mask=m)
            return cur + plsc.all_reduce_population_count(m)[0]

        lax.fori_loop(0, CAP // 16, l1_inner, b * SBC)
        return carry

    lax.fori_loop(0, SB, l1_outer, 0)

    lanes16 = lax.iota(jnp.int32, 16)

    def l2_outer(r, carry):
        cur2, curv = carry
        vb = (r // 16) * SBC

        def l2_inner(v, c2):
            dv = sb_dst[pl.ds(vb + v * 16, 16)]
            sv = sb_src[pl.ds(vb + v * 16, 16)]
            m = dv == r
            plsc.store_compressed(sels2_v.at[pl.ds(c2, 16)], sv, mask=m)
            return c2 + plsc.all_reduce_population_count(m)[0]

        c2 = lax.fori_loop(0, SBC // 16, l2_inner, cur2)
        padn = (16 - (c2 & 15)) & 15
        plsc.store_compressed(sels2_v.at[pl.ds(c2, 16)], zrowv,
                              mask=lanes16 < padn)
        c2 = c2 + padn
        nvec = (c2 - cur2) >> 4
        plsc.store_compressed(rowd_v.at[pl.ds(curv, 16)],
                              jnp.full((16,), 0, jnp.int32) + r,
                              mask=lanes16 < nvec)
        return (c2, curv + nvec)

    lax.fori_loop(0, RT, l2_outer, (jnp.int32(0), jnp.int32(0)))

    pltpu.sync_copy(sels_v.at[pl.ds(0, CAP)], sels_hbm.at[pl.ds(wid * CAP, CAP)])
    pltpu.sync_copy(seld_v.at[pl.ds(0, CAP)], seld_hbm.at[pl.ds(wid * CAP, CAP)])
    pltpu.sync_copy(selt_v.at[pl.ds(0, CAP)], selt_hbm.at[pl.ds(wid * CAP, CAP)])
    pltpu.sync_copy(deg_l.at[pl.ds(0, RT * T)], deg_hbm.at[pl.ds(lo * T, RT * T)])
    pltpu.sync_copy(sels2_v.at[pl.ds(0, CAP2)],
                    sels2_hbm.at[pl.ds(wid * CAP2, CAP2)])
    pltpu.sync_copy(rowd_v.at[pl.ds(0, VEC2)],
                    rowd_hbm.at[pl.ds(wid * VEC2, VEC2)])


@functools.cache
def _sc_u_kernel():
    return pl.kernel(
        _sc_u_body,
        out_type=jax.ShapeDtypeStruct((NP, T), jnp.float32),
        mesh=_mesh(),
        scratch_types=[
            pltpu.VMEM((NP,), jnp.float32),
            pltpu.VMEM((CAP,), jnp.int32),
            pltpu.VMEM((CAP,), jnp.int32),
            pltpu.VMEM((CAP,), jnp.int32),
            pltpu.VMEM((ACC_R, T), jnp.float32),
        ],
    )


def _sc_u_body(r_hbm, sels_hbm, seld_hbm, selt_hbm,
               u_hbm,
               r_v, sels_v, seld_v, selt_v, u_l):
    c = lax.axis_index("c")
    s = lax.axis_index("s")
    wid = c * 16 + s
    lo = wid * RT
    pltpu.sync_copy(r_hbm, r_v)
    pltpu.sync_copy(sels_hbm.at[pl.ds(wid * CAP, CAP)], sels_v)
    pltpu.sync_copy(seld_hbm.at[pl.ds(wid * CAP, CAP)], seld_v)
    pltpu.sync_copy(selt_hbm.at[pl.ds(wid * CAP, CAP)], selt_v)

    zf = jnp.zeros((16,), jnp.float32)

    def zrow(i, carry):
        u_l[i, :] = zf
        return carry

    lax.fori_loop(0, ACC_R, zrow, 0)

    lanes = lax.iota(jnp.int32, 16)

    def body(i, carry):
        svec = sels_v[pl.ds(i * 16, 16)]
        dvec = seld_v[pl.ds(i * 16, 16)]
        tvec = selt_v[pl.ds(i * 16, 16)]
        for k in range(16):
            rs = r_v[pl.ds(svec[k], 16)][0]
            u_l[dvec[k], :] = (u_l[dvec[k], :]
                               + jnp.where(lanes == tvec[k], rs, 0.0))
        return carry

    lax.fori_loop(0, CAP // 16, body, 0)
    pltpu.sync_copy(u_l.at[pl.ds(0, RT)], u_hbm.at[pl.ds(lo, RT)])


@functools.cache
def _sc_spmm_kernel():
    return pl.kernel(
        _sc_spmm_body,
        out_type=jax.ShapeDtypeStruct((NP, D), jnp.float32),
        mesh=_mesh(),
        scratch_types=[
            pltpu.VMEM((G2, 128), jnp.int32),
            pltpu.VMEM((VEC2 + 16,), jnp.int32),
            pltpu.VMEM((2, 128, D), jnp.float32),
            pltpu.VMEM((ACC_R, D), jnp.float32),
            pltpu.SemaphoreType.DMA,
            pltpu.SemaphoreType.DMA,
        ],
    )


def _sc_spmm_body(g_hbm, sels2_hbm, rowd_hbm, z128_hbm,
                  p0_hbm,
                  sels_v, rowd_v, rows_v, acc, sem0, sem1):
    c = lax.axis_index("c")
    s = lax.axis_index("s")
    wid = c * 16 + s
    lo = wid * RT
    pltpu.sync_copy(sels2_hbm.at[pl.ds(wid * G2, G2)], sels_v)
    pltpu.sync_copy(rowd_hbm.at[pl.ds(wid * VEC2, VEC2)],
                    rowd_v.at[pl.ds(0, VEC2)])
    pltpu.sync_copy(z128_hbm, acc.at[pl.ds(0, 128)])
    pltpu.sync_copy(z128_hbm, acc.at[pl.ds(128, 128)])
    pltpu.sync_copy(z128_hbm.at[pl.ds(0, ACC_R - 256)],
                    acc.at[pl.ds(256, ACC_R - 256)])

    def issue(k, buf, sem):
        for w in range(8):
            idxv = sels_v[k, pl.ds(w * 16, 16)]
            pltpu.async_copy(g_hbm.at[idxv],
                             rows_v.at[buf].at[pl.ds(w * 16, 16)], sem)

    def drain(buf, sem):
        pltpu.make_async_copy(g_hbm.at[sels_v.at[0]],
                              rows_v.at[buf], sem).wait()

    def accum(k, buf):
        dvec8 = rowd_v[pl.ds(k * 8, 16)]
        for w in range(8):
            d = dvec8[w]
            for p in range(8):
                v = [rows_v[buf, w * 16 + i, pl.ds(p * 16, 16)]
                     for i in range(16)]
                while len(v) > 1:
                    v = [v[2 * j] + v[2 * j + 1] for j in range(len(v) // 2)]
                plsc.addupdate(acc.at[d, pl.ds(p * 16, 16)], v[0])

    issue(0, 0, sem0)

    def body(m, carry):
        k0 = 2 * m
        drain(0, sem0)
        issue(k0 + 1, 1, sem1)
        accum(k0, 0)
        drain(1, sem1)

        @pl.when(k0 + 2 < G2)
        def _():
            issue(k0 + 2, 0, sem0)

        accum(k0 + 1, 1)
        return carry

    lax.fori_loop(0, G2 // 2, body, 0)
    pltpu.sync_copy(acc.at[pl.ds(0, RT)], p0_hbm.at[pl.ds(lo, RT)])


def _row_mask():
    rows = (pl.program_id(0) * BLK
            + lax.broadcasted_iota(jnp.int32, (BLK, 1), 0))
    return rows < N


def _tc_prep_body(h_ref, deg16_ref, r_ref, g0_ref):
    hr = h_ref[...]
    deg = jnp.maximum(deg16_ref[...][:, 0], 1.0)
    r = lax.rsqrt(deg)
    nrm = jnp.sqrt(jnp.sum(hr * hr, axis=1, keepdims=True))
    h0 = hr * jnp.minimum(1.0, 1.0 / (nrm + 1e-7))
    r_ref[...] = r
    g0_ref[...] = jnp.where(_row_mask(), h0 * r[:, None], 0.0)


def _tc_prep(hraw, deg16):
    grid = NP // BLK
    return pl.pallas_call(
        _tc_prep_body,
        grid=(grid,),
        in_specs=[pl.BlockSpec((BLK, D), lambda i: (i, 0)),
                  pl.BlockSpec((BLK, T), lambda i: (i, 0))],
        out_specs=[pl.BlockSpec((BLK,), lambda i: (i,)),
                   pl.BlockSpec((BLK, D), lambda i: (i, 0))],
        out_shape=[jax.ShapeDtypeStruct((NP,), jnp.float32),
                   jax.ShapeDtypeStruct((NP, D), jnp.float32)],
    )(hraw, deg16)


def _tc_c_body(u_ref, r_ref, et_ref, we1_ref, b1_ref, we2_ref, b2_ref,
               c1_ref, c2_ref):
    hi = lax.Precision.HIGHEST
    v1 = jnp.dot(et_ref[...], we1_ref[...], precision=hi,
                 preferred_element_type=jnp.float32) + b1_ref[...][None, :]
    v2 = jnp.dot(et_ref[...], we2_ref[...], precision=hi,
                 preferred_element_type=jnp.float32) + b2_ref[...][None, :]
    u = u_ref[...]
    r = r_ref[...][:, None]
    c1_ref[...] = jnp.dot(u, v1, precision=hi,
                          preferred_element_type=jnp.float32) * r
    c2_ref[...] = jnp.dot(u, v2, precision=hi,
                          preferred_element_type=jnp.float32) * r


def _tc_c(u, r, et, we1, b1, we2, b2):
    grid = NP // BLK
    return pl.pallas_call(
        _tc_c_body,
        grid=(grid,),
        in_specs=[pl.BlockSpec((BLK, T), lambda i: (i, 0)),
                  pl.BlockSpec((BLK,), lambda i: (i,)),
                  pl.BlockSpec((T, T), lambda i: (0, 0)),
                  pl.BlockSpec((T, D), lambda i: (0, 0)),
                  pl.BlockSpec((D,), lambda i: (0,)),
                  pl.BlockSpec((T, D), lambda i: (0, 0)),
                  pl.BlockSpec((D,), lambda i: (0,))],
        out_specs=[pl.BlockSpec((BLK, D), lambda i: (i, 0)),
                   pl.BlockSpec((BLK, D), lambda i: (i, 0))],
        out_shape=[jax.ShapeDtypeStruct((NP, D), jnp.float32),
                   jax.ShapeDtypeStruct((NP, D), jnp.float32)],
    )(u, r, et, we1, b1, we2, b2)


def _tc_layer_body(p0_ref, r_ref, c_ref, w_ref, out_ref, *, act, emit_g):
    r = r_ref[...][:, None]
    accv = p0_ref[...] * r
    z = jnp.dot(accv, w_ref[...], precision=lax.Precision.HIGHEST,
                preferred_element_type=jnp.float32) + c_ref[...]
    if act:
        z = jnp.where(z > 0.0, z, jnp.exp(jnp.minimum(z, 0.0)) - 1.0)
    if emit_g:
        z = jnp.where(_row_mask(), z * r, 0.0)
    out_ref[...] = z


def _tc_layer(p0, r, cc, wx, act, emit_g):
    grid = NP // BLK
    return pl.pallas_call(
        functools.partial(_tc_layer_body, act=act, emit_g=emit_g),
        grid=(grid,),
        in_specs=[pl.BlockSpec((BLK, D), lambda i: (i, 0)),
                  pl.BlockSpec((BLK,), lambda i: (i,)),
                  pl.BlockSpec((BLK, D), lambda i: (i, 0)),
                  pl.BlockSpec((D, D), lambda i: (0, 0))],
        out_specs=pl.BlockSpec((BLK, D), lambda i: (i, 0)),
        out_shape=jax.ShapeDtypeStruct((NP, D), jnp.float32),
    )(p0, r, cc, wx)


def kernel(x, edge_index, edge_attr, node_table, edge_table,
           W1, b1, W2, b2, slices):
    f32 = jnp.float32
    src = edge_index[0].astype(jnp.int32)
    dst = edge_index[1].astype(jnp.int32)
    typ = edge_attr[:, 0].astype(jnp.int32)
    xi = x[:, 0].astype(jnp.int32)
    e = src.shape[0]
    src_p = jnp.concatenate([src, jnp.zeros((EPAD - e,), jnp.int32)])
    dst_p = jnp.concatenate([dst, jnp.full((EPAD - e,), NP, jnp.int32)])
    typ_p = jnp.concatenate([typ, jnp.zeros((EPAD - e,), jnp.int32)])
    xi_p = jnp.concatenate(
        [xi, jnp.zeros((XG * 128 - N,), jnp.int32)]).reshape(32, XG // 32, 128)
    z128 = jnp.zeros((128, D), f32)

    hraw = _sc_embed_kernel()(node_table, xi_p)
    sels, seld, selt, degf, sels2, rowd = _sc_prep_kernel()(src_p, dst_p, typ_p)
    r, g = _tc_prep(hraw[:NP], degf.reshape(NP, T))
    u = _sc_u_kernel()(r, sels, seld, selt)
    c1, c2 = _tc_c(u, r, edge_table, W1[D:], b1, W2[D:], b2)
    wx1, wx2 = W1[:D], W2[:D]

    h = g
    for layer in range(6):
        p0 = _sc_spmm_kernel()(g, sels2.reshape(32 * G2, 128), rowd, z128)
        if layer % 2 == 0:
            g = _tc_layer(p0, r, c1, wx1, act=True, emit_g=True)
        elif layer < 5:
            g = _tc_layer(p0, r, c2, wx2, act=False, emit_g=True)
        else:
            h = _tc_layer(p0, r, c2, wx2, act=False, emit_g=False)

    out = h[:N].reshape(N // 1000, 1000, D)
    return out * jnp.asarray(slices // 1000, dtype=out.dtype)


# speedup vs baseline: 11.2642x; 11.2606x over previous
import functools

import jax
import jax.numpy as jnp
from jax import lax
from jax.experimental import pallas as pl
from jax.experimental.pallas import tpu as pltpu
from jax.experimental.pallas import tpu_sc as plsc

N = 10000
D = 128
T = 16
NP = 10240
RT = NP // 32
DROW = RT
ACC_R = RT + 8
EG = 2560
EPAD = EG * 128
CH = 32
NCH = EG // CH
CAP = 12288
CAPG = CAP // 128
XG = 96
BLK = 1024
CAP2 = 16384
VEC2 = CAP2 // 16
G2 = CAP2 // 128
SB = 20
SBC = 1040
ZROW = N + 200

_SEL = 32 * CAP


def _mesh():
    return plsc.VectorSubcoreMesh(
        core_axis_name="c", subcore_axis_name="s",
        num_cores=2, num_subcores=16)


@functools.cache
def _sc_embed_kernel():
    return pl.kernel(
        _sc_embed_body,
        out_type=jax.ShapeDtypeStruct((XG * 128, D), jnp.float32),
        mesh=_mesh(),
        scratch_types=[
            pltpu.VMEM((3, 128), jnp.int32),
            pltpu.VMEM((128, D), jnp.float32),
            pltpu.SemaphoreType.DMA,
        ],
    )


def _sc_embed_body(table_hbm, xi_hbm, hraw_hbm, xi_v, rows_v, sem):
    c = lax.axis_index("c")
    s = lax.axis_index("s")
    wid = c * 16 + s
    pltpu.sync_copy(xi_hbm.at[wid], xi_v)
    for j in range(3):
        pltpu.async_copy(table_hbm.at[xi_v.at[j]], rows_v, sem).wait()
        pltpu.sync_copy(rows_v, hraw_hbm.at[pl.ds((wid * 3 + j) * 128, 128)])


@functools.cache
def _sc_prep_kernel():
    return pl.kernel(
        _sc_prep_body,
        out_type=(jax.ShapeDtypeStruct((_SEL,), jnp.int32),
                  jax.ShapeDtypeStruct((_SEL,), jnp.int32),
                  jax.ShapeDtypeStruct((_SEL,), jnp.int32),
                  jax.ShapeDtypeStruct((NP * T,), jnp.float32),
                  jax.ShapeDtypeStruct((32 * CAP2,), jnp.int32),
                  jax.ShapeDtypeStruct((32 * VEC2,), jnp.int32)),
        mesh=_mesh(),
        compiler_params=pltpu.CompilerParams(needs_layout_passes=False),
        scratch_types=[
            pltpu.VMEM((CH * 128,), jnp.int32),
            pltpu.VMEM((CH * 128,), jnp.int32),
            pltpu.VMEM((CH * 128,), jnp.int32),
            pltpu.VMEM((CAP + 16,), jnp.int32),
            pltpu.VMEM((CAP + 16,), jnp.int32),
            pltpu.VMEM((CAP + 16,), jnp.int32),
            pltpu.VMEM((ACC_R * T,), jnp.float32),
            pltpu.VMEM((SB * SBC,), jnp.int32),
            pltpu.VMEM((SB * SBC,), jnp.int32),
            pltpu.VMEM((CAP2 + 16,), jnp.int32),
            pltpu.VMEM((VEC2 + 16,), jnp.int32),
        ],
    )


def _sc_prep_body(src_hbm, dst_hbm, typ_hbm,
                  sels_hbm, seld_hbm, selt_hbm, deg_hbm, sels2_hbm, rowd_hbm,
                  src_v, dst_v, typ_v, sels_v, seld_v, selt_v, deg_l,
                  sb_src, sb_dst, sels2_v, rowd_v):
    c = lax.axis_index("c")
    s = lax.axis_index("s")
    wid = c * 16 + s
    lo = wid * RT

    zv = jnp.zeros((16,), jnp.int32)
    dv = jnp.full((16,), DROW, jnp.int32)
    spreadv = N + lax.iota(jnp.int32, 16)

    def fill(i, carry):
        sels_v[pl.ds(i * 16, 16)] = spreadv + (i % 14) * 16
        seld_v[pl.ds(i * 16, 16)] = dv
        selt_v[pl.ds(i * 16, 16)] = zv
        return carry

    lax.fori_loop(0, (CAP + 16) // 16, fill, 0)

    zf = jnp.zeros((16,), jnp.float32)

    def zrow(i, carry):
        deg_l[pl.ds(i * 16, 16)] = zf
        return carry

    lax.fori_loop(0, ACC_R * T // 16, zrow, 0)

    def chunk(ci, cur):
        pltpu.sync_copy(src_hbm.at[pl.ds(ci * CH * 128, CH * 128)], src_v)
        pltpu.sync_copy(dst_hbm.at[pl.ds(ci * CH * 128, CH * 128)], dst_v)
        pltpu.sync_copy(typ_hbm.at[pl.ds(ci * CH * 128, CH * 128)], typ_v)
        for v in range(CH * 8):
            dsts = dst_v[pl.ds(v * 16, 16)]
            srcs = src_v[pl.ds(v * 16, 16)]
            typs = typ_v[pl.ds(v * 16, 16)]
            m = (dsts >= lo) & (dsts < lo + RT)
            plsc.store_compressed(sels_v.at[pl.ds(cur, 16)], srcs, mask=m)
            plsc.store_compressed(seld_v.at[pl.ds(cur, 16)], dsts - lo, mask=m)
            plsc.store_compressed(selt_v.at[pl.ds(cur, 16)], typs, mask=m)
            cnt = plsc.all_reduce_population_count(m)[0]
            cur = cur + cnt
        return cur

    lax.fori_loop(0, NCH, chunk, jnp.int32(0))

    e0 = jnp.where(lax.iota(jnp.int32, 16) == 0, 1.0, 0.0)

    def dbody(i, carry):
        dvec = seld_v[pl.ds(i * 16, 16)]
        for k in range(16):
            d = dvec[k]
            deg_l[pl.ds(d * T, 16)] = deg_l[pl.ds(d * T, 16)] + e0
        return carry

    lax.fori_loop(0, CAP // 16, dbody, 0)

    sent = jnp.full((16,), 30000, jnp.int32)
    zrowv = jnp.full((16,), ZROW, jnp.int32)
    drowv = jnp.full((16,), DROW, jnp.int32)

    def sfill(i, carry):
        sb_dst[pl.ds(i * 16, 16)] = sent
        return carry

    lax.fori_loop(0, SB * SBC // 16, sfill, 0)

    def s2fill(i, carry):
        sels2_v[pl.ds(i * 16, 16)] = zrowv
        return carry

    lax.fori_loop(0, (CAP2 + 16) // 16, s2fill, 0)

    def rdfill(i, carry):
        rowd_v[pl.ds(i * 16, 16)] = drowv
        return carry

    lax.fori_loop(0, (VEC2 + 16) // 16, rdfill, 0)

    def l1_outer(b, carry):
        def l1_inner(v, cur):
            dv = seld_v[pl.ds(v * 16, 16)]
            sv = sels_v[pl.ds(v * 16, 16)]
            m = (dv >= b * 16) & (dv < b * 16 + 16)
            plsc.store_compressed(sb_src.at[pl.ds(cur, 16)], sv, mask=m)
            plsc.store_compressed(sb_dst.at[pl.ds(cur, 16)], dv, mask=m)
            return cur + plsc.all_reduce_population_count(m)[0]

        lax.fori_loop(0, CAP // 16, l1_inner, b * SBC)
        return carry

    lax.fori_loop(0, SB, l1_outer, 0)

    lanes16 = lax.iota(jnp.int32, 16)

    def l2_outer(r, carry):
        cur2, curv = carry
        vb = (r // 16) * SBC

        def l2_inner(v, c2):
            dv = sb_dst[pl.ds(vb + v * 16, 16)]
            sv = sb_src[pl.ds(vb + v * 16, 16)]
            m = dv == r
            plsc.store_compressed(sels2_v.at[pl.ds(c2, 16)], sv, mask=m)
            return c2 + plsc.all_reduce_population_count(m)[0]

        c2 = lax.fori_loop(0, SBC // 16, l2_inner, cur2)
        padn = (16 - (c2 & 15)) & 15
        plsc.store_compressed(sels2_v.at[pl.ds(c2, 16)], zrowv,
                              mask=lanes16 < padn)
        c2 = c2 + padn
        nvec = (c2 - cur2) >> 4
        plsc.store_compressed(rowd_v.at[pl.ds(curv, 16)],
                              jnp.full((16,), 0, jnp.int32) + r,
                              mask=lanes16 < nvec)
        return (c2, curv + nvec)

    lax.fori_loop(0, RT, l2_outer, (jnp.int32(0), jnp.int32(0)))

    pltpu.sync_copy(sels_v.at[pl.ds(0, CAP)], sels_hbm.at[pl.ds(wid * CAP, CAP)])
    pltpu.sync_copy(seld_v.at[pl.ds(0, CAP)], seld_hbm.at[pl.ds(wid * CAP, CAP)])
    pltpu.sync_copy(selt_v.at[pl.ds(0, CAP)], selt_hbm.at[pl.ds(wid * CAP, CAP)])
    pltpu.sync_copy(deg_l.at[pl.ds(0, RT * T)], deg_hbm.at[pl.ds(lo * T, RT * T)])
    pltpu.sync_copy(sels2_v.at[pl.ds(0, CAP2)],
                    sels2_hbm.at[pl.ds(wid * CAP2, CAP2)])
    pltpu.sync_copy(rowd_v.at[pl.ds(0, VEC2)],
                    rowd_hbm.at[pl.ds(wid * VEC2, VEC2)])


@functools.cache
def _sc_u_kernel():
    return pl.kernel(
        _sc_u_body,
        out_type=jax.ShapeDtypeStruct((NP, T), jnp.float32),
        mesh=_mesh(),
        scratch_types=[
            pltpu.VMEM((NP + 16,), jnp.float32),
            pltpu.VMEM((CAP,), jnp.int32),
            pltpu.VMEM((CAP,), jnp.int32),
            pltpu.VMEM((CAP,), jnp.int32),
            pltpu.VMEM((ACC_R, T), jnp.float32),
        ],
    )


def _sc_u_body(r_hbm, sels_hbm, seld_hbm, selt_hbm,
               u_hbm,
               r_v, sels_v, seld_v, selt_v, u_l):
    c = lax.axis_index("c")
    s = lax.axis_index("s")
    wid = c * 16 + s
    lo = wid * RT
    pltpu.sync_copy(r_hbm, r_v.at[pl.ds(0, NP)])
    pltpu.sync_copy(sels_hbm.at[pl.ds(wid * CAP, CAP)], sels_v)
    pltpu.sync_copy(seld_hbm.at[pl.ds(wid * CAP, CAP)], seld_v)
    pltpu.sync_copy(selt_hbm.at[pl.ds(wid * CAP, CAP)], selt_v)

    zf = jnp.zeros((16,), jnp.float32)

    def zrow(i, carry):
        u_l[i, :] = zf
        return carry

    lax.fori_loop(0, ACC_R, zrow, 0)

    lanes = lax.iota(jnp.int32, 16)

    def body(i, carry):
        svec = sels_v[pl.ds(i * 16, 16)]
        dvec = seld_v[pl.ds(i * 16, 16)]
        tvec = selt_v[pl.ds(i * 16, 16)]
        for k in range(16):
            rs = r_v[pl.ds(svec[k], 16)][0]
            u_l[dvec[k], :] = (u_l[dvec[k], :]
                               + jnp.where(lanes == tvec[k], rs, 0.0))
        return carry

    lax.fori_loop(0, CAP // 16, body, 0)
    pltpu.sync_copy(u_l.at[pl.ds(0, RT)], u_hbm.at[pl.ds(lo, RT)])


@functools.cache
def _sc_spmm_kernel():
    return pl.kernel(
        _sc_spmm_body,
        out_type=jax.ShapeDtypeStruct((NP, D), jnp.float32),
        mesh=_mesh(),
        scratch_types=[
            pltpu.VMEM((CAP,), jnp.int32),
            pltpu.VMEM((CAP,), jnp.int32),
            pltpu.VMEM((2, 128, D), jnp.float32),
            pltpu.VMEM((ACC_R, D), jnp.float32),
            pltpu.SemaphoreType.DMA,
            pltpu.SemaphoreType.DMA,
        ],
    )


def _sc_spmm_body(g_hbm, sels_hbm, seld_hbm, z128_hbm,
                  p0_hbm,
                  sels_v, seld_v, rows_v, acc, sem0, sem1):
    c = lax.axis_index("c")
    s = lax.axis_index("s")
    wid = c * 16 + s
    lo = wid * RT
    pltpu.sync_copy(sels_hbm.at[pl.ds(wid * CAP, CAP)], sels_v)
    pltpu.sync_copy(seld_hbm.at[pl.ds(wid * CAP, CAP)], seld_v)
    pltpu.sync_copy(z128_hbm, acc.at[pl.ds(0, 128)])
    pltpu.sync_copy(z128_hbm, acc.at[pl.ds(128, 128)])
    pltpu.sync_copy(z128_hbm.at[pl.ds(0, ACC_R - 256)],
                    acc.at[pl.ds(256, ACC_R - 256)])

    def issue(k, buf, sem):
        pltpu.async_copy(g_hbm.at[sels_v.at[pl.ds(k * 128, 128)]],
                         rows_v.at[buf], sem)

    def drain(buf, sem):
        pltpu.make_async_copy(g_hbm.at[sels_v.at[pl.ds(0, 128)]],
                              rows_v.at[buf], sem).wait()

    def accum(k, buf):
        for w in range(8):
            dvec = seld_v[pl.ds(k * 128 + w * 16, 16)]
            for i in range(16):
                d = dvec[i]
                for p in range(8):
                    plsc.addupdate(acc.at[d, pl.ds(p * 16, 16)],
                                   rows_v[buf, w * 16 + i, pl.ds(p * 16, 16)])

    issue(0, 0, sem0)

    def body(m, carry):
        k0 = 2 * m
        drain(0, sem0)
        issue(k0 + 1, 1, sem1)
        accum(k0, 0)
        drain(1, sem1)

        @pl.when(k0 + 2 < CAPG)
        def _():
            issue(k0 + 2, 0, sem0)

        accum(k0 + 1, 1)
        return carry

    lax.fori_loop(0, CAPG // 2, body, 0)
    pltpu.sync_copy(acc.at[pl.ds(0, RT)], p0_hbm.at[pl.ds(lo, RT)])


def _row_mask():
    rows = (pl.program_id(0) * BLK
            + lax.broadcasted_iota(jnp.int32, (BLK, 1), 0))
    return rows < N


def _tc_prep_body(h_ref, deg16_ref, r_ref, g0_ref):
    hr = h_ref[...]
    deg = jnp.maximum(deg16_ref[...][:, 0], 1.0)
    r = lax.rsqrt(deg)
    nrm = jnp.sqrt(jnp.sum(hr * hr, axis=1, keepdims=True))
    h0 = hr * jnp.minimum(1.0, 1.0 / (nrm + 1e-7))
    r_ref[...] = r
    g0_ref[...] = jnp.where(_row_mask(), h0 * r[:, None], 0.0)


def _tc_prep(hraw, deg16):
    grid = NP // BLK
    return pl.pallas_call(
        _tc_prep_body,
        grid=(grid,),
        in_specs=[pl.BlockSpec((BLK, D), lambda i: (i, 0)),
                  pl.BlockSpec((BLK, T), lambda i: (i, 0))],
        out_specs=[pl.BlockSpec((BLK,), lambda i: (i,)),
                   pl.BlockSpec((BLK, D), lambda i: (i, 0))],
        out_shape=[jax.ShapeDtypeStruct((NP,), jnp.float32),
                   jax.ShapeDtypeStruct((NP, D), jnp.float32)],
    )(hraw, deg16)


def _tc_c_body(u_ref, r_ref, et_ref, we1_ref, b1_ref, we2_ref, b2_ref,
               c1_ref, c2_ref):
    hi = lax.Precision.HIGHEST
    v1 = jnp.dot(et_ref[...], we1_ref[...], precision=hi,
                 preferred_element_type=jnp.float32) + b1_ref[...][None, :]
    v2 = jnp.dot(et_ref[...], we2_ref[...], precision=hi,
                 preferred_element_type=jnp.float32) + b2_ref[...][None, :]
    u = u_ref[...]
    r = r_ref[...][:, None]
    c1_ref[...] = jnp.dot(u, v1, precision=hi,
                          preferred_element_type=jnp.float32) * r
    c2_ref[...] = jnp.dot(u, v2, precision=hi,
                          preferred_element_type=jnp.float32) * r


def _tc_c(u, r, et, we1, b1, we2, b2):
    grid = NP // BLK
    return pl.pallas_call(
        _tc_c_body,
        grid=(grid,),
        in_specs=[pl.BlockSpec((BLK, T), lambda i: (i, 0)),
                  pl.BlockSpec((BLK,), lambda i: (i,)),
                  pl.BlockSpec((T, T), lambda i: (0, 0)),
                  pl.BlockSpec((T, D), lambda i: (0, 0)),
                  pl.BlockSpec((D,), lambda i: (0,)),
                  pl.BlockSpec((T, D), lambda i: (0, 0)),
                  pl.BlockSpec((D,), lambda i: (0,))],
        out_specs=[pl.BlockSpec((BLK, D), lambda i: (i, 0)),
                   pl.BlockSpec((BLK, D), lambda i: (i, 0))],
        out_shape=[jax.ShapeDtypeStruct((NP, D), jnp.float32),
                   jax.ShapeDtypeStruct((NP, D), jnp.float32)],
    )(u, r, et, we1, b1, we2, b2)


def _tc_layer_body(p0_ref, r_ref, c_ref, w_ref, out_ref, *, act, emit_g):
    r = r_ref[...][:, None]
    accv = p0_ref[...] * r
    z = jnp.dot(accv, w_ref[...], precision=lax.Precision.HIGHEST,
                preferred_element_type=jnp.float32) + c_ref[...]
    if act:
        z = jnp.where(z > 0.0, z, jnp.exp(jnp.minimum(z, 0.0)) - 1.0)
    if emit_g:
        z = jnp.where(_row_mask(), z * r, 0.0)
    out_ref[...] = z


def _tc_layer(p0, r, cc, wx, act, emit_g):
    grid = NP // BLK
    return pl.pallas_call(
        functools.partial(_tc_layer_body, act=act, emit_g=emit_g),
        grid=(grid,),
        in_specs=[pl.BlockSpec((BLK, D), lambda i: (i, 0)),
                  pl.BlockSpec((BLK,), lambda i: (i,)),
                  pl.BlockSpec((BLK, D), lambda i: (i, 0)),
                  pl.BlockSpec((D, D), lambda i: (0, 0))],
        out_specs=pl.BlockSpec((BLK, D), lambda i: (i, 0)),
        out_shape=jax.ShapeDtypeStruct((NP, D), jnp.float32),
    )(p0, r, cc, wx)


def kernel(x, edge_index, edge_attr, node_table, edge_table,
           W1, b1, W2, b2, slices):
    f32 = jnp.float32
    src = edge_index[0].astype(jnp.int32)
    dst = edge_index[1].astype(jnp.int32)
    typ = edge_attr[:, 0].astype(jnp.int32)
    xi = x[:, 0].astype(jnp.int32)
    e = src.shape[0]
    src_p = jnp.concatenate([src, jnp.zeros((EPAD - e,), jnp.int32)])
    dst_p = jnp.concatenate([dst, jnp.full((EPAD - e,), NP, jnp.int32)])
    typ_p = jnp.concatenate([typ, jnp.zeros((EPAD - e,), jnp.int32)])
    xi_p = jnp.concatenate(
        [xi, jnp.zeros((XG * 128 - N,), jnp.int32)]).reshape(32, XG // 32, 128)
    z128 = jnp.zeros((128, D), f32)

    hraw = _sc_embed_kernel()(node_table, xi_p)
    sels, seld, selt, degf, sels2, rowd = _sc_prep_kernel()(src_p, dst_p, typ_p)
    r, g = _tc_prep(hraw[:NP], degf.reshape(NP, T))
    u = _sc_u_kernel()(r, sels, seld, selt)
    c1, c2 = _tc_c(u, r, edge_table, W1[D:], b1, W2[D:], b2)
    wx1, wx2 = W1[:D], W2[:D]

    h = g
    for layer in range(6):
        p0 = _sc_spmm_kernel()(g, sels, seld, z128)
        if layer % 2 == 0:
            g = _tc_layer(p0, r, c1, wx1, act=True, emit_g=True)
        elif layer < 5:
            g = _tc_layer(p0, r, c2, wx2, act=False, emit_g=True)
        else:
            h = _tc_layer(p0, r, c2, wx2, act=False, emit_g=False)

    out = h[:N].reshape(N // 1000, 1000, D)
    return out * jnp.asarray(slices // 1000, dtype=out.dtype)


# speedup vs baseline: 12.5336x; 1.1127x over previous
import functools

import jax
import jax.numpy as jnp
from jax import lax
from jax.experimental import pallas as pl
from jax.experimental.pallas import tpu as pltpu
from jax.experimental.pallas import tpu_sc as plsc

N = 10000
D = 128
T = 16
NP = 10240
RT = NP // 32
DROW = RT
ACC_R = RT + 8
EG = 2560
EPAD = EG * 128
CH = 32
NCH = EG // CH
CAP = 12288
CAPG = CAP // 128
XG = 96
BLK = 1024
CAP2 = 16384
VEC2 = CAP2 // 16
G2 = CAP2 // 128
SB = 20
SBC = 1040
ZROW = N + 200

_SEL = 32 * CAP


def _mesh():
    return plsc.VectorSubcoreMesh(
        core_axis_name="c", subcore_axis_name="s",
        num_cores=2, num_subcores=16)


@functools.cache
def _sc_embed_kernel():
    return pl.kernel(
        _sc_embed_body,
        out_type=jax.ShapeDtypeStruct((XG * 128, D), jnp.float32),
        mesh=_mesh(),
        scratch_types=[
            pltpu.VMEM((3, 128), jnp.int32),
            pltpu.VMEM((128, D), jnp.float32),
            pltpu.SemaphoreType.DMA,
        ],
    )


def _sc_embed_body(table_hbm, xi_hbm, hraw_hbm, xi_v, rows_v, sem):
    c = lax.axis_index("c")
    s = lax.axis_index("s")
    wid = c * 16 + s
    pltpu.sync_copy(xi_hbm.at[wid], xi_v)
    for j in range(3):
        pltpu.async_copy(table_hbm.at[xi_v.at[j]], rows_v, sem).wait()
        pltpu.sync_copy(rows_v, hraw_hbm.at[pl.ds((wid * 3 + j) * 128, 128)])


@functools.cache
def _sc_prep_kernel():
    return pl.kernel(
        _sc_prep_body,
        out_type=(jax.ShapeDtypeStruct((_SEL,), jnp.int32),
                  jax.ShapeDtypeStruct((_SEL,), jnp.int32),
                  jax.ShapeDtypeStruct((_SEL,), jnp.int32),
                  jax.ShapeDtypeStruct((NP * T,), jnp.float32)),
        mesh=_mesh(),
        compiler_params=pltpu.CompilerParams(needs_layout_passes=False),
        scratch_types=[
            pltpu.VMEM((CH * 128,), jnp.int32),
            pltpu.VMEM((CH * 128,), jnp.int32),
            pltpu.VMEM((CH * 128,), jnp.int32),
            pltpu.VMEM((CAP + 16,), jnp.int32),
            pltpu.VMEM((CAP + 16,), jnp.int32),
            pltpu.VMEM((CAP + 16,), jnp.int32),
            pltpu.VMEM((ACC_R * T,), jnp.float32),
        ],
    )


def _sc_prep_body(src_hbm, dst_hbm, typ_hbm,
                  sels_hbm, seld_hbm, selt_hbm, deg_hbm,
                  src_v, dst_v, typ_v, sels_v, seld_v, selt_v, deg_l):
    c = lax.axis_index("c")
    s = lax.axis_index("s")
    wid = c * 16 + s
    lo = wid * RT

    zv = jnp.zeros((16,), jnp.int32)
    dv = jnp.full((16,), DROW, jnp.int32)
    spreadv = N + lax.iota(jnp.int32, 16)

    def fill(i, carry):
        sels_v[pl.ds(i * 16, 16)] = spreadv + (i % 14) * 16
        seld_v[pl.ds(i * 16, 16)] = dv
        selt_v[pl.ds(i * 16, 16)] = zv
        return carry

    lax.fori_loop(0, (CAP + 16) // 16, fill, 0)

    zf = jnp.zeros((16,), jnp.float32)

    def zrow(i, carry):
        deg_l[pl.ds(i * 16, 16)] = zf
        return carry

    lax.fori_loop(0, ACC_R * T // 16, zrow, 0)

    def chunk(ci, cur):
        pltpu.sync_copy(src_hbm.at[pl.ds(ci * CH * 128, CH * 128)], src_v)
        pltpu.sync_copy(dst_hbm.at[pl.ds(ci * CH * 128, CH * 128)], dst_v)
        pltpu.sync_copy(typ_hbm.at[pl.ds(ci * CH * 128, CH * 128)], typ_v)
        for v in range(CH * 8):
            dsts = dst_v[pl.ds(v * 16, 16)]
            srcs = src_v[pl.ds(v * 16, 16)]
            typs = typ_v[pl.ds(v * 16, 16)]
            m = (dsts >= lo) & (dsts < lo + RT)
            plsc.store_compressed(sels_v.at[pl.ds(cur, 16)], srcs, mask=m)
            plsc.store_compressed(seld_v.at[pl.ds(cur, 16)], dsts - lo, mask=m)
            plsc.store_compressed(selt_v.at[pl.ds(cur, 16)], typs, mask=m)
            cnt = plsc.all_reduce_population_count(m)[0]
            cur = cur + cnt
        return cur

    lax.fori_loop(0, NCH, chunk, jnp.int32(0))

    e0 = jnp.where(lax.iota(jnp.int32, 16) == 0, 1.0, 0.0)

    def dbody(i, carry):
        dvec = seld_v[pl.ds(i * 16, 16)]
        for k in range(16):
            d = dvec[k]
            deg_l[pl.ds(d * T, 16)] = deg_l[pl.ds(d * T, 16)] + e0
        return carry

    lax.fori_loop(0, CAP // 16, dbody, 0)

    pltpu.sync_copy(sels_v.at[pl.ds(0, CAP)], sels_hbm.at[pl.ds(wid * CAP, CAP)])
    pltpu.sync_copy(seld_v.at[pl.ds(0, CAP)], seld_hbm.at[pl.ds(wid * CAP, CAP)])
    pltpu.sync_copy(selt_v.at[pl.ds(0, CAP)], selt_hbm.at[pl.ds(wid * CAP, CAP)])
    pltpu.sync_copy(deg_l.at[pl.ds(0, RT * T)], deg_hbm.at[pl.ds(lo * T, RT * T)])


@functools.cache
def _sc_u_kernel():
    return pl.kernel(
        _sc_u_body,
        out_type=jax.ShapeDtypeStruct((NP, T), jnp.float32),
        mesh=_mesh(),
        scratch_types=[
            pltpu.VMEM((NP + 16,), jnp.float32),
            pltpu.VMEM((CAP,), jnp.int32),
            pltpu.VMEM((CAP,), jnp.int32),
            pltpu.VMEM((CAP,), jnp.int32),
            pltpu.VMEM((ACC_R, T), jnp.float32),
        ],
    )


def _sc_u_body(r_hbm, sels_hbm, seld_hbm, selt_hbm,
               u_hbm,
               r_v, sels_v, seld_v, selt_v, u_l):
    c = lax.axis_index("c")
    s = lax.axis_index("s")
    wid = c * 16 + s
    lo = wid * RT
    pltpu.sync_copy(r_hbm, r_v.at[pl.ds(0, NP)])
    pltpu.sync_copy(sels_hbm.at[pl.ds(wid * CAP, CAP)], sels_v)
    pltpu.sync_copy(seld_hbm.at[pl.ds(wid * CAP, CAP)], seld_v)
    pltpu.sync_copy(selt_hbm.at[pl.ds(wid * CAP, CAP)], selt_v)

    zf = jnp.zeros((16,), jnp.float32)

    def zrow(i, carry):
        u_l[i, :] = zf
        return carry

    lax.fori_loop(0, ACC_R, zrow, 0)

    lanes = lax.iota(jnp.int32, 16)

    def body(i, carry):
        svec = sels_v[pl.ds(i * 16, 16)]
        dvec = seld_v[pl.ds(i * 16, 16)]
        tvec = selt_v[pl.ds(i * 16, 16)]
        for k in range(16):
            rs = r_v[pl.ds(svec[k], 16)][0]
            u_l[dvec[k], :] = (u_l[dvec[k], :]
                               + jnp.where(lanes == tvec[k], rs, 0.0))
        return carry

    lax.fori_loop(0, CAP // 16, body, 0)
    pltpu.sync_copy(u_l.at[pl.ds(0, RT)], u_hbm.at[pl.ds(lo, RT)])


@functools.cache
def _sc_spmm_kernel():
    return pl.kernel(
        _sc_spmm_body,
        out_type=jax.ShapeDtypeStruct((NP, D), jnp.float32),
        mesh=_mesh(),
        scratch_types=[
            pltpu.VMEM((CAP,), jnp.int32),
            pltpu.VMEM((CAP,), jnp.int32),
            pltpu.VMEM((2, 128, D), jnp.float32),
            pltpu.VMEM((ACC_R, D), jnp.float32),
            pltpu.SemaphoreType.DMA,
            pltpu.SemaphoreType.DMA,
        ],
    )


def _sc_spmm_body(g_hbm, sels_hbm, seld_hbm, z128_hbm,
                  p0_hbm,
                  sels_v, seld_v, rows_v, acc, sem0, sem1):
    c = lax.axis_index("c")
    s = lax.axis_index("s")
    wid = c * 16 + s
    lo = wid * RT
    pltpu.sync_copy(sels_hbm.at[pl.ds(wid * CAP, CAP)], sels_v)
    pltpu.sync_copy(seld_hbm.at[pl.ds(wid * CAP, CAP)], seld_v)
    pltpu.sync_copy(z128_hbm, acc.at[pl.ds(0, 128)])
    pltpu.sync_copy(z128_hbm, acc.at[pl.ds(128, 128)])
    pltpu.sync_copy(z128_hbm.at[pl.ds(0, ACC_R - 256)],
                    acc.at[pl.ds(256, ACC_R - 256)])

    def issue(k, buf, sem):
        pltpu.async_copy(g_hbm.at[sels_v.at[pl.ds(k * 128, 128)]],
                         rows_v.at[buf], sem)

    def drain(buf, sem):
        pltpu.make_async_copy(g_hbm.at[sels_v.at[pl.ds(0, 128)]],
                              rows_v.at[buf], sem).wait()

    def accum(k, buf):
        for w in range(8):
            dvec = seld_v[pl.ds(k * 128 + w * 16, 16)]
            for i in range(16):
                d = dvec[i]
                for p in range(8):
                    plsc.addupdate(acc.at[d, pl.ds(p * 16, 16)],
                                   rows_v[buf, w * 16 + i, pl.ds(p * 16, 16)])

    issue(0, 0, sem0)

    def body(m, carry):
        k0 = 2 * m
        drain(0, sem0)
        issue(k0 + 1, 1, sem1)
        accum(k0, 0)
        drain(1, sem1)

        @pl.when(k0 + 2 < CAPG)
        def _():
            issue(k0 + 2, 0, sem0)

        accum(k0 + 1, 1)
        return carry

    lax.fori_loop(0, CAPG // 2, body, 0)
    pltpu.sync_copy(acc.at[pl.ds(0, RT)], p0_hbm.at[pl.ds(lo, RT)])


def _row_mask():
    rows = (pl.program_id(0) * BLK
            + lax.broadcasted_iota(jnp.int32, (BLK, 1), 0))
    return rows < N


def _tc_prep_body(h_ref, deg16_ref, r_ref, g0_ref):
    hr = h_ref[...]
    deg = jnp.maximum(deg16_ref[...][:, 0], 1.0)
    r = lax.rsqrt(deg)
    nrm = jnp.sqrt(jnp.sum(hr * hr, axis=1, keepdims=True))
    h0 = hr * jnp.minimum(1.0, 1.0 / (nrm + 1e-7))
    r_ref[...] = r
    g0_ref[...] = jnp.where(_row_mask(), h0 * r[:, None], 0.0)


def _tc_prep(hraw, deg16):
    grid = NP // BLK
    return pl.pallas_call(
        _tc_prep_body,
        grid=(grid,),
        in_specs=[pl.BlockSpec((BLK, D), lambda i: (i, 0)),
                  pl.BlockSpec((BLK, T), lambda i: (i, 0))],
        out_specs=[pl.BlockSpec((BLK,), lambda i: (i,)),
                   pl.BlockSpec((BLK, D), lambda i: (i, 0))],
        out_shape=[jax.ShapeDtypeStruct((NP,), jnp.float32),
                   jax.ShapeDtypeStruct((NP, D), jnp.float32)],
    )(hraw, deg16)


def _tc_c_body(u_ref, r_ref, et_ref, we1_ref, b1_ref, we2_ref, b2_ref,
               c1_ref, c2_ref):
    hi = lax.Precision.HIGHEST
    v1 = jnp.dot(et_ref[...], we1_ref[...], precision=hi,
                 preferred_element_type=jnp.float32) + b1_ref[...][None, :]
    v2 = jnp.dot(et_ref[...], we2_ref[...], precision=hi,
                 preferred_element_type=jnp.float32) + b2_ref[...][None, :]
    u = u_ref[...]
    r = r_ref[...][:, None]
    c1_ref[...] = jnp.dot(u, v1, precision=hi,
                          preferred_element_type=jnp.float32) * r
    c2_ref[...] = jnp.dot(u, v2, precision=hi,
                          preferred_element_type=jnp.float32) * r


def _tc_c(u, r, et, we1, b1, we2, b2):
    grid = NP // BLK
    return pl.pallas_call(
        _tc_c_body,
        grid=(grid,),
        in_specs=[pl.BlockSpec((BLK, T), lambda i: (i, 0)),
                  pl.BlockSpec((BLK,), lambda i: (i,)),
                  pl.BlockSpec((T, T), lambda i: (0, 0)),
                  pl.BlockSpec((T, D), lambda i: (0, 0)),
                  pl.BlockSpec((D,), lambda i: (0,)),
                  pl.BlockSpec((T, D), lambda i: (0, 0)),
                  pl.BlockSpec((D,), lambda i: (0,))],
        out_specs=[pl.BlockSpec((BLK, D), lambda i: (i, 0)),
                   pl.BlockSpec((BLK, D), lambda i: (i, 0))],
        out_shape=[jax.ShapeDtypeStruct((NP, D), jnp.float32),
                   jax.ShapeDtypeStruct((NP, D), jnp.float32)],
    )(u, r, et, we1, b1, we2, b2)


def _tc_layer_body(p0_ref, r_ref, c_ref, w_ref, out_ref, *, act, emit_g):
    r = r_ref[...][:, None]
    accv = p0_ref[...] * r
    z = jnp.dot(accv, w_ref[...], precision=lax.Precision.HIGHEST,
                preferred_element_type=jnp.float32) + c_ref[...]
    if act:
        z = jnp.where(z > 0.0, z, jnp.exp(jnp.minimum(z, 0.0)) - 1.0)
    if emit_g:
        z = jnp.where(_row_mask(), z * r, 0.0)
    out_ref[...] = z


def _tc_layer(p0, r, cc, wx, act, emit_g):
    grid = NP // BLK
    return pl.pallas_call(
        functools.partial(_tc_layer_body, act=act, emit_g=emit_g),
        grid=(grid,),
        in_specs=[pl.BlockSpec((BLK, D), lambda i: (i, 0)),
                  pl.BlockSpec((BLK,), lambda i: (i,)),
                  pl.BlockSpec((BLK, D), lambda i: (i, 0)),
                  pl.BlockSpec((D, D), lambda i: (0, 0))],
        out_specs=pl.BlockSpec((BLK, D), lambda i: (i, 0)),
        out_shape=jax.ShapeDtypeStruct((NP, D), jnp.float32),
    )(p0, r, cc, wx)


def kernel(x, edge_index, edge_attr, node_table, edge_table,
           W1, b1, W2, b2, slices):
    f32 = jnp.float32
    src = edge_index[0].astype(jnp.int32)
    dst = edge_index[1].astype(jnp.int32)
    typ = edge_attr[:, 0].astype(jnp.int32)
    xi = x[:, 0].astype(jnp.int32)
    e = src.shape[0]
    src_p = jnp.concatenate([src, jnp.zeros((EPAD - e,), jnp.int32)])
    dst_p = jnp.concatenate([dst, jnp.full((EPAD - e,), NP, jnp.int32)])
    typ_p = jnp.concatenate([typ, jnp.zeros((EPAD - e,), jnp.int32)])
    xi_p = jnp.concatenate(
        [xi, jnp.zeros((XG * 128 - N,), jnp.int32)]).reshape(32, XG // 32, 128)
    z128 = jnp.zeros((128, D), f32)

    hraw = _sc_embed_kernel()(node_table, xi_p)
    sels, seld, selt, degf = _sc_prep_kernel()(src_p, dst_p, typ_p)
    r, g = _tc_prep(hraw[:NP], degf.reshape(NP, T))
    u = _sc_u_kernel()(r, sels, seld, selt)
    c1, c2 = _tc_c(u, r, edge_table, W1[D:], b1, W2[D:], b2)
    wx1, wx2 = W1[:D], W2[:D]

    h = g
    for layer in range(6):
        p0 = _sc_spmm_kernel()(g, sels, seld, z128)
        if layer % 2 == 0:
            g = _tc_layer(p0, r, c1, wx1, act=True, emit_g=True)
        elif layer < 5:
            g = _tc_layer(p0, r, c2, wx2, act=False, emit_g=True)
        else:
            h = _tc_layer(p0, r, c2, wx2, act=False, emit_g=False)

    out = h[:N].reshape(N // 1000, 1000, D)
    return out * jnp.asarray(slices // 1000, dtype=out.dtype)


# speedup vs baseline: 14.2280x; 1.1352x over previous
import functools

import jax
import jax.numpy as jnp
from jax import lax
from jax.experimental import pallas as pl
from jax.experimental.pallas import tpu as pltpu
from jax.experimental.pallas import tpu_sc as plsc

N = 10000
D = 128
T = 16
NP = 10240
RT = NP // 32
DROW = RT
ACC_R = RT + 8
EG = 2560
EPAD = EG * 128
CH = 32
NCH = EG // CH
CAP = 12288
CAPG = CAP // 128
XG = 96
BLK = 1024
CAP2 = 16384
VEC2 = CAP2 // 16
G2 = CAP2 // 128
SB = 20
SBC = 1040
ZROW = N + 200

_SEL = 32 * CAP


def _mesh():
    return plsc.VectorSubcoreMesh(
        core_axis_name="c", subcore_axis_name="s",
        num_cores=2, num_subcores=16)


@functools.cache
def _sc_embed_kernel():
    return pl.kernel(
        _sc_embed_body,
        out_type=jax.ShapeDtypeStruct((XG * 128, D), jnp.float32),
        mesh=_mesh(),
        scratch_types=[
            pltpu.VMEM((3, 128), jnp.int32),
            pltpu.VMEM((128, D), jnp.float32),
            pltpu.SemaphoreType.DMA,
        ],
    )


def _sc_embed_body(table_hbm, xi_hbm, hraw_hbm, xi_v, rows_v, sem):
    c = lax.axis_index("c")
    s = lax.axis_index("s")
    wid = c * 16 + s
    pltpu.sync_copy(xi_hbm.at[wid], xi_v)
    for j in range(3):
        pltpu.async_copy(table_hbm.at[xi_v.at[j]], rows_v, sem).wait()
        pltpu.sync_copy(rows_v, hraw_hbm.at[pl.ds((wid * 3 + j) * 128, 128)])


@functools.cache
def _sc_prep_kernel():
    return pl.kernel(
        _sc_prep_body,
        out_type=(jax.ShapeDtypeStruct((_SEL,), jnp.int32),
                  jax.ShapeDtypeStruct((_SEL,), jnp.int32),
                  jax.ShapeDtypeStruct((_SEL,), jnp.int32),
                  jax.ShapeDtypeStruct((NP * T,), jnp.float32),
                  jax.ShapeDtypeStruct((32 * 16,), jnp.int32)),
        mesh=_mesh(),
        compiler_params=pltpu.CompilerParams(needs_layout_passes=False),
        scratch_types=[
            pltpu.VMEM((CH * 128,), jnp.int32),
            pltpu.VMEM((CH * 128,), jnp.int32),
            pltpu.VMEM((CH * 128,), jnp.int32),
            pltpu.VMEM((CAP + 16,), jnp.int32),
            pltpu.VMEM((CAP + 16,), jnp.int32),
            pltpu.VMEM((CAP + 16,), jnp.int32),
            pltpu.VMEM((ACC_R * T,), jnp.float32),
            pltpu.VMEM((16,), jnp.int32),
        ],
    )


def _sc_prep_body(src_hbm, dst_hbm, typ_hbm,
                  sels_hbm, seld_hbm, selt_hbm, deg_hbm, cnt_hbm,
                  src_v, dst_v, typ_v, sels_v, seld_v, selt_v, deg_l, cnt_v):
    c = lax.axis_index("c")
    s = lax.axis_index("s")
    wid = c * 16 + s
    lo = wid * RT

    zv = jnp.zeros((16,), jnp.int32)
    dv = jnp.full((16,), DROW, jnp.int32)
    spreadv = N + lax.iota(jnp.int32, 16)

    def fill(i, carry):
        sels_v[pl.ds(i * 16, 16)] = spreadv + (i % 14) * 16
        seld_v[pl.ds(i * 16, 16)] = dv
        selt_v[pl.ds(i * 16, 16)] = zv
        return carry

    lax.fori_loop(0, (CAP + 16) // 16, fill, 0)

    zf = jnp.zeros((16,), jnp.float32)

    def zrow(i, carry):
        deg_l[pl.ds(i * 16, 16)] = zf
        return carry

    lax.fori_loop(0, ACC_R * T // 16, zrow, 0)

    def chunk(ci, cur):
        pltpu.sync_copy(src_hbm.at[pl.ds(ci * CH * 128, CH * 128)], src_v)
        pltpu.sync_copy(dst_hbm.at[pl.ds(ci * CH * 128, CH * 128)], dst_v)
        pltpu.sync_copy(typ_hbm.at[pl.ds(ci * CH * 128, CH * 128)], typ_v)
        for v in range(CH * 8):
            dsts = dst_v[pl.ds(v * 16, 16)]
            srcs = src_v[pl.ds(v * 16, 16)]
            typs = typ_v[pl.ds(v * 16, 16)]
            m = (dsts >= lo) & (dsts < lo + RT)
            plsc.store_compressed(sels_v.at[pl.ds(cur, 16)], srcs, mask=m)
            plsc.store_compressed(seld_v.at[pl.ds(cur, 16)], dsts - lo, mask=m)
            plsc.store_compressed(selt_v.at[pl.ds(cur, 16)], typs, mask=m)
            cnt = plsc.all_reduce_population_count(m)[0]
            cur = cur + cnt
        return cur

    nsel = lax.fori_loop(0, NCH, chunk, jnp.int32(0))
    cnt_v[pl.ds(0, 16)] = jnp.full((16,), 0, jnp.int32) + nsel

    e0 = jnp.where(lax.iota(jnp.int32, 16) == 0, 1.0, 0.0)

    def dbody(i, carry):
        dvec = seld_v[pl.ds(i * 16, 16)]
        for k in range(16):
            d = dvec[k]
            deg_l[pl.ds(d * T, 16)] = deg_l[pl.ds(d * T, 16)] + e0
        return carry

    lax.fori_loop(0, CAP // 16, dbody, 0)

    pltpu.sync_copy(sels_v.at[pl.ds(0, CAP)], sels_hbm.at[pl.ds(wid * CAP, CAP)])
    pltpu.sync_copy(seld_v.at[pl.ds(0, CAP)], seld_hbm.at[pl.ds(wid * CAP, CAP)])
    pltpu.sync_copy(selt_v.at[pl.ds(0, CAP)], selt_hbm.at[pl.ds(wid * CAP, CAP)])
    pltpu.sync_copy(deg_l.at[pl.ds(0, RT * T)], deg_hbm.at[pl.ds(lo * T, RT * T)])
    pltpu.sync_copy(cnt_v, cnt_hbm.at[pl.ds(wid * 16, 16)])


@functools.cache
def _sc_u_kernel():
    return pl.kernel(
        _sc_u_body,
        out_type=jax.ShapeDtypeStruct((NP, T), jnp.float32),
        mesh=_mesh(),
        scratch_types=[
            pltpu.VMEM((NP + 16,), jnp.float32),
            pltpu.VMEM((CAP,), jnp.int32),
            pltpu.VMEM((CAP,), jnp.int32),
            pltpu.VMEM((CAP,), jnp.int32),
            pltpu.VMEM((ACC_R, T), jnp.float32),
        ],
    )


def _sc_u_body(r_hbm, sels_hbm, seld_hbm, selt_hbm,
               u_hbm,
               r_v, sels_v, seld_v, selt_v, u_l):
    c = lax.axis_index("c")
    s = lax.axis_index("s")
    wid = c * 16 + s
    lo = wid * RT
    pltpu.sync_copy(r_hbm, r_v.at[pl.ds(0, NP)])
    pltpu.sync_copy(sels_hbm.at[pl.ds(wid * CAP, CAP)], sels_v)
    pltpu.sync_copy(seld_hbm.at[pl.ds(wid * CAP, CAP)], seld_v)
    pltpu.sync_copy(selt_hbm.at[pl.ds(wid * CAP, CAP)], selt_v)

    zf = jnp.zeros((16,), jnp.float32)

    def zrow(i, carry):
        u_l[i, :] = zf
        return carry

    lax.fori_loop(0, ACC_R, zrow, 0)

    lanes = lax.iota(jnp.int32, 16)

    def body(i, carry):
        svec = sels_v[pl.ds(i * 16, 16)]
        dvec = seld_v[pl.ds(i * 16, 16)]
        tvec = selt_v[pl.ds(i * 16, 16)]
        for k in range(16):
            rs = r_v[pl.ds(svec[k], 16)][0]
            u_l[dvec[k], :] = (u_l[dvec[k], :]
                               + jnp.where(lanes == tvec[k], rs, 0.0))
        return carry

    lax.fori_loop(0, CAP // 16, body, 0)
    pltpu.sync_copy(u_l.at[pl.ds(0, RT)], u_hbm.at[pl.ds(lo, RT)])


@functools.cache
def _sc_spmm_kernel():
    return pl.kernel(
        _sc_spmm_body,
        out_type=jax.ShapeDtypeStruct((NP, D), jnp.float32),
        mesh=_mesh(),
        scratch_types=[
            pltpu.VMEM((CAP,), jnp.int32),
            pltpu.VMEM((CAP,), jnp.int32),
            pltpu.VMEM((2, 128, D), jnp.float32),
            pltpu.VMEM((ACC_R, D), jnp.float32),
            pltpu.VMEM((16,), jnp.int32),
            pltpu.SemaphoreType.DMA,
            pltpu.SemaphoreType.DMA,
        ],
    )


def _sc_spmm_body(g_hbm, sels_hbm, seld_hbm, z128_hbm, cnt_hbm,
                  p0_hbm,
                  sels_v, seld_v, rows_v, acc, cnt_v, sem0, sem1):
    c = lax.axis_index("c")
    s = lax.axis_index("s")
    wid = c * 16 + s
    lo = wid * RT
    pltpu.sync_copy(sels_hbm.at[pl.ds(wid * CAP, CAP)], sels_v)
    pltpu.sync_copy(seld_hbm.at[pl.ds(wid * CAP, CAP)], seld_v)
    pltpu.sync_copy(cnt_hbm.at[pl.ds(wid * 16, 16)], cnt_v)
    pltpu.sync_copy(z128_hbm, acc.at[pl.ds(0, 128)])
    pltpu.sync_copy(z128_hbm, acc.at[pl.ds(128, 128)])
    pltpu.sync_copy(z128_hbm.at[pl.ds(0, ACC_R - 256)],
                    acc.at[pl.ds(256, ACC_R - 256)])

    def issue(k, buf, sem):
        pltpu.async_copy(g_hbm.at[sels_v.at[pl.ds(k * 128, 128)]],
                         rows_v.at[buf], sem)

    def drain(buf, sem):
        pltpu.make_async_copy(g_hbm.at[sels_v.at[pl.ds(0, 128)]],
                              rows_v.at[buf], sem).wait()

    def accum(k, buf):
        for w in range(8):
            dvec = seld_v[pl.ds(k * 128 + w * 16, 16)]
            for i in range(16):
                d = dvec[i]
                for p in range(8):
                    plsc.addupdate(acc.at[d, pl.ds(p * 16, 16)],
                                   rows_v[buf, w * 16 + i, pl.ds(p * 16, 16)])

    nsel = cnt_v[pl.ds(0, 16)][0]
    niter = (nsel + 255) >> 8

    issue(0, 0, sem0)

    def body(m, carry):
        k0 = 2 * m
        drain(0, sem0)
        issue(k0 + 1, 1, sem1)
        accum(k0, 0)
        drain(1, sem1)

        @pl.when(k0 + 2 < 2 * niter)
        def _():
            issue(k0 + 2, 0, sem0)

        accum(k0 + 1, 1)
        return carry

    lax.fori_loop(0, niter, body, 0)
    pltpu.sync_copy(acc.at[pl.ds(0, RT)], p0_hbm.at[pl.ds(lo, RT)])


def _row_mask():
    rows = (pl.program_id(0) * BLK
            + lax.broadcasted_iota(jnp.int32, (BLK, 1), 0))
    return rows < N


def _tc_prep_body(h_ref, deg16_ref, r_ref, g0_ref):
    hr = h_ref[...]
    deg = jnp.maximum(deg16_ref[...][:, 0], 1.0)
    r = lax.rsqrt(deg)
    nrm = jnp.sqrt(jnp.sum(hr * hr, axis=1, keepdims=True))
    h0 = hr * jnp.minimum(1.0, 1.0 / (nrm + 1e-7))
    r_ref[...] = r
    g0_ref[...] = jnp.where(_row_mask(), h0 * r[:, None], 0.0)


def _tc_prep(hraw, deg16):
    grid = NP // BLK
    return pl.pallas_call(
        _tc_prep_body,
        grid=(grid,),
        in_specs=[pl.BlockSpec((BLK, D), lambda i: (i, 0)),
                  pl.BlockSpec((BLK, T), lambda i: (i, 0))],
        out_specs=[pl.BlockSpec((BLK,), lambda i: (i,)),
                   pl.BlockSpec((BLK, D), lambda i: (i, 0))],
        out_shape=[jax.ShapeDtypeStruct((NP,), jnp.float32),
                   jax.ShapeDtypeStruct((NP, D), jnp.float32)],
    )(hraw, deg16)


def _tc_c_body(u_ref, r_ref, et_ref, we1_ref, b1_ref, we2_ref, b2_ref,
               c1_ref, c2_ref):
    hi = lax.Precision.HIGHEST
    v1 = jnp.dot(et_ref[...], we1_ref[...], precision=hi,
                 preferred_element_type=jnp.float32) + b1_ref[...][None, :]
    v2 = jnp.dot(et_ref[...], we2_ref[...], precision=hi,
                 preferred_element_type=jnp.float32) + b2_ref[...][None, :]
    u = u_ref[...]
    r = r_ref[...][:, None]
    c1_ref[...] = jnp.dot(u, v1, precision=hi,
                          preferred_element_type=jnp.float32) * r
    c2_ref[...] = jnp.dot(u, v2, precision=hi,
                          preferred_element_type=jnp.float32) * r


def _tc_c(u, r, et, we1, b1, we2, b2):
    grid = NP // BLK
    return pl.pallas_call(
        _tc_c_body,
        grid=(grid,),
        in_specs=[pl.BlockSpec((BLK, T), lambda i: (i, 0)),
                  pl.BlockSpec((BLK,), lambda i: (i,)),
                  pl.BlockSpec((T, T), lambda i: (0, 0)),
                  pl.BlockSpec((T, D), lambda i: (0, 0)),
                  pl.BlockSpec((D,), lambda i: (0,)),
                  pl.BlockSpec((T, D), lambda i: (0, 0)),
                  pl.BlockSpec((D,), lambda i: (0,))],
        out_specs=[pl.BlockSpec((BLK, D), lambda i: (i, 0)),
                   pl.BlockSpec((BLK, D), lambda i: (i, 0))],
        out_shape=[jax.ShapeDtypeStruct((NP, D), jnp.float32),
                   jax.ShapeDtypeStruct((NP, D), jnp.float32)],
    )(u, r, et, we1, b1, we2, b2)


def _tc_layer_body(p0_ref, r_ref, c_ref, w_ref, out_ref, *, act, emit_g):
    r = r_ref[...][:, None]
    accv = p0_ref[...] * r
    z = jnp.dot(accv, w_ref[...], precision=lax.Precision.HIGHEST,
                preferred_element_type=jnp.float32) + c_ref[...]
    if act:
        z = jnp.where(z > 0.0, z, jnp.exp(jnp.minimum(z, 0.0)) - 1.0)
    if emit_g:
        z = jnp.where(_row_mask(), z * r, 0.0)
    out_ref[...] = z


def _tc_layer(p0, r, cc, wx, act, emit_g):
    grid = NP // BLK
    return pl.pallas_call(
        functools.partial(_tc_layer_body, act=act, emit_g=emit_g),
        grid=(grid,),
        in_specs=[pl.BlockSpec((BLK, D), lambda i: (i, 0)),
                  pl.BlockSpec((BLK,), lambda i: (i,)),
                  pl.BlockSpec((BLK, D), lambda i: (i, 0)),
                  pl.BlockSpec((D, D), lambda i: (0, 0))],
        out_specs=pl.BlockSpec((BLK, D), lambda i: (i, 0)),
        out_shape=jax.ShapeDtypeStruct((NP, D), jnp.float32),
    )(p0, r, cc, wx)


def kernel(x, edge_index, edge_attr, node_table, edge_table,
           W1, b1, W2, b2, slices):
    f32 = jnp.float32
    src = edge_index[0].astype(jnp.int32)
    dst = edge_index[1].astype(jnp.int32)
    typ = edge_attr[:, 0].astype(jnp.int32)
    xi = x[:, 0].astype(jnp.int32)
    e = src.shape[0]
    src_p = jnp.concatenate([src, jnp.zeros((EPAD - e,), jnp.int32)])
    dst_p = jnp.concatenate([dst, jnp.full((EPAD - e,), NP, jnp.int32)])
    typ_p = jnp.concatenate([typ, jnp.zeros((EPAD - e,), jnp.int32)])
    xi_p = jnp.concatenate(
        [xi, jnp.zeros((XG * 128 - N,), jnp.int32)]).reshape(32, XG // 32, 128)
    z128 = jnp.zeros((128, D), f32)

    hraw = _sc_embed_kernel()(node_table, xi_p)
    sels, seld, selt, degf, cnt = _sc_prep_kernel()(src_p, dst_p, typ_p)
    r, g = _tc_prep(hraw[:NP], degf.reshape(NP, T))
    u = _sc_u_kernel()(r, sels, seld, selt)
    c1, c2 = _tc_c(u, r, edge_table, W1[D:], b1, W2[D:], b2)
    wx1, wx2 = W1[:D], W2[:D]

    h = g
    for layer in range(6):
        p0 = _sc_spmm_kernel()(g, sels, seld, z128, cnt)
        if layer % 2 == 0:
            g = _tc_layer(p0, r, c1, wx1, act=True, emit_g=True)
        elif layer < 5:
            g = _tc_layer(p0, r, c2, wx2, act=False, emit_g=True)
        else:
            h = _tc_layer(p0, r, c2, wx2, act=False, emit_g=False)

    out = h[:N].reshape(N // 1000, 1000, D)
    return out * jnp.asarray(slices // 1000, dtype=out.dtype)
